# baseline (speedup 1.0000x reference)
import functools

import jax
import jax.numpy as jnp
from jax import lax
from jax.experimental import pallas as pl
from jax.experimental.pallas import tpu as pltpu
from jax.experimental.pallas import tpu_sc as plsc

N = 10000
D = 128
H = 16
NC = 2
NS = 16
NW = NC * NS
L = 16
CI = 80
NI = 1
CH = CI * NI
NBUF = 5
NP = 10240
ROWS_PER_TILE = NP // NS


@functools.lru_cache(maxsize=None)
def _make_spmm(kch: int, fuse_relu: bool):
    mesh = plsc.VectorSubcoreMesh(core_axis_name="c", subcore_axis_name="s")

    def body(table_hbm, b_hbm, src_hbm, dst_hbm, val_hbm, zeros_hbm, out_hbm,
             src_v, dst_v, val_v, rows_v, msg_v, pa_v, pb_v, b_v,
             sg0, sg1, sg2, sg3, sg4, ss0, ss1, ss2, ss3, ss4, accum, table_s):
        c = lax.axis_index("c")
        s = lax.axis_index("s")
        w = c * NS + s
        sems_g = (sg0, sg1, sg2, sg3, sg4)
        sems_s = (ss0, ss1, ss2, ss3, ss4)
        tslice = pl.ds(s * ROWS_PER_TILE, ROWS_PER_TILE)

        scope_pro = jax.named_scope("prologue")
        scope_pro.__enter__()
        dz = pltpu.async_copy(zeros_hbm, accum.at[tslice], ss0)
        if fuse_relu:
            da = pltpu.async_copy(table_hbm.at[0, tslice], pa_v, sg0)
            db = pltpu.async_copy(table_hbm.at[1, tslice], pb_v, sg1)
        else:
            dt = None
        d1 = pltpu.async_copy(src_hbm.at[w], src_v, ss1)
        d2 = pltpu.async_copy(dst_hbm.at[w], dst_v, ss2)
        d3 = pltpu.async_copy(val_hbm.at[w], val_v, ss3)
        if fuse_relu:
            pltpu.sync_copy(b_hbm, b_v)
            da.wait()
            db.wait()
            bb = b_v[0, :]

            def relu_row(r, carry):
                pa_v[r, :] = jnp.maximum(pa_v[r, :] + pb_v[r, :] + bb, 0.0)
                return carry

            lax.fori_loop(0, ROWS_PER_TILE, relu_row, 0)
            pltpu.sync_copy(pa_v, table_s.at[tslice])
        dz.wait()
        d1.wait()
        d2.wait()
        d3.wait()
        plsc.subcore_barrier()
        scope_pro.__exit__(None, None, None)
        scope_main = jax.named_scope("edge_loop")
        gather_src = table_s if fuse_relu else table_hbm

        def g_start(k, b):
            for h in range(NI):
                pltpu.async_copy(gather_src.at[dst_v.at[k, h]],
                                 rows_v.at[b, pl.ds(h * CI, CI)], sems_g[b])

        def g_wait(k, b):
            for h in range(NI):
                pltpu.make_async_copy(gather_src.at[dst_v.at[k, h]],
                                      rows_v.at[b, pl.ds(h * CI, CI)],
                                      sems_g[b]).wait()

        def s_start(k, b):
            for h in range(NI):
                pltpu.async_copy(msg_v.at[b, pl.ds(h * CI, CI)],
                                 accum.at[src_v.at[k, h]], sems_s[b],
                                 add=True)

        def s_wait(k, b):
            for h in range(NI):
                pltpu.make_async_copy(msg_v.at[b, pl.ds(h * CI, CI)],
                                      accum.at[src_v.at[k, h]],
                                      sems_s[b]).wait()

        def compute(k, b):
            for g in range(CH // L):
                vv = val_v[k, pl.ds(g * L, L)]
                for j in range(L):
                    i = g * L + j
                    msg_v[b, i, :] = rows_v[b, i, :] * vv[j]

        scope_main.__enter__()
        for b in range(NBUF):
            g_start(b, b)

        def outer(kk, carry):
            for b in range(NBUF):
                k = kk * NBUF + b
                g_wait(k, b)

                @pl.when(k >= NBUF)
                def _():
                    s_wait(k - NBUF, b)

                compute(k, b)
                g_start(k + NBUF, b)
                s_start(k, b)
            return carry

        lax.fori_loop(0, (kch - NBUF) // NBUF, outer, 0)
        for b in range(NBUF):
            k = kch - NBUF + b
            g_wait(k, b)
            s_wait(k - NBUF, b)
            compute(k, b)
            s_start(k, b)
        for b in range(NBUF):
            s_wait(kch - NBUF + b, b)
        scope_main.__exit__(None, None, None)
        with jax.named_scope("writeout"):
            plsc.subcore_barrier()
            pltpu.sync_copy(accum.at[tslice], out_hbm.at[c, tslice])

    prt = ROWS_PER_TILE if fuse_relu else 8
    return pl.kernel(
        body,
        out_type=jax.ShapeDtypeStruct((NC, NP, H), jnp.float32),
        mesh=mesh,
        scratch_types=[
            pltpu.VMEM((kch, NI, CI), jnp.int32),
            pltpu.VMEM((kch, NI, CI), jnp.int32),
            pltpu.VMEM((kch, CH), jnp.float32),
            pltpu.VMEM((NBUF, CH, H), jnp.float32),
            pltpu.VMEM((NBUF, CH, H), jnp.float32),
            pltpu.VMEM((prt, H), jnp.float32),
            pltpu.VMEM((prt, H), jnp.float32),
            pltpu.VMEM((1, H), jnp.float32),
            pltpu.SemaphoreType.DMA,
            pltpu.SemaphoreType.DMA,
            pltpu.SemaphoreType.DMA,
            pltpu.SemaphoreType.DMA,
            pltpu.SemaphoreType.DMA,
            pltpu.SemaphoreType.DMA,
            pltpu.SemaphoreType.DMA,
            pltpu.SemaphoreType.DMA,
            pltpu.SemaphoreType.DMA,
            pltpu.SemaphoreType.DMA,
            pltpu.VMEM_SHARED((NP, H), jnp.float32),
            pltpu.VMEM_SHARED((NP, H), jnp.float32),
        ],
        compiler_params=pltpu.CompilerParams(use_tc_tiling_on_sc=False),
    )


def _mm1_body(x_ref, w_ref, o_ref):
    o_ref[:N, :] = jnp.dot(x_ref[:, :], w_ref[:, :],
                           preferred_element_type=jnp.float32)


def _relu_body(p_ref, b_ref, o_ref):
    o_ref[:, :] = jnp.maximum(p_ref[0] + p_ref[1] + b_ref[:, :], 0.0)


def _out_body(p_ref, w_ref, b_ref, o_ref):
    a = p_ref[0] + p_ref[1]
    t = jnp.dot(a, w_ref[:, :], preferred_element_type=jnp.float32)
    o_ref[:, :] = jax.nn.sigmoid(t + b_ref[:, :])[:N, :]


def kernel(x, lap_indices, lap_values, W1, b1, W2, b2):
    E = lap_values.shape[0]
    src = lap_indices[0].astype(jnp.int32)
    dst = lap_indices[1].astype(jnp.int32)

    kch = -(-E // (NW * CH))
    kch = ((kch + NBUF - 1) // NBUF) * NBUF
    e_pad = NW * kch * CH
    padn = e_pad - E
    if padn:
        pad_idx = (jnp.arange(padn, dtype=jnp.int32) * 61) % N
        src = jnp.concatenate([src, pad_idx])
        dst = jnp.concatenate([dst, pad_idx])
        lap_values = jnp.concatenate(
            [lap_values, jnp.zeros((padn,), jnp.float32)])
    src3 = src.reshape(NW, kch, NI, CI)
    dst3 = dst.reshape(NW, kch, NI, CI)
    val3 = lap_values.reshape(NW, kch, CH)
    zeros = jnp.zeros((ROWS_PER_TILE, H), jnp.float32)

    spmm1 = _make_spmm(kch, False)
    spmm2 = _make_spmm(kch, True)

    t1 = pl.pallas_call(
        _mm1_body,
        out_shape=jax.ShapeDtypeStruct((NP, H), jnp.float32),
    )(x, W1)

    b1r = b1.reshape(1, H)
    p1 = spmm1(t1, b1r, src3, dst3, val3, zeros)
    p2 = spmm2(p1, b1r, src3, dst3, val3, zeros)

    out = pl.pallas_call(
        _out_body,
        out_shape=jax.ShapeDtypeStruct((N, 1), jnp.float32),
    )(p2, W2, b2.reshape(1, 1))
    return out


# speedup vs baseline: 1.0976x; 1.0976x over previous
import functools

import jax
import jax.numpy as jnp
from jax import lax
from jax.experimental import pallas as pl
from jax.experimental.pallas import tpu as pltpu
from jax.experimental.pallas import tpu_sc as plsc

N = 10000
D = 128
H = 16
NC = 2
NS = 16
NW = NC * NS
L = 16
CI = 80
NI = 1
CH = CI * NI
NBUF = 5
NP = 10240
ROWS_PER_TILE = NP // NS


@functools.lru_cache(maxsize=None)
def _make_spmm(kch: int, fuse_relu: bool):
    mesh = plsc.VectorSubcoreMesh(core_axis_name="c", subcore_axis_name="s")

    def body(table_hbm, b_hbm, src_hbm, dst_hbm, val_hbm, zeros_hbm, out_hbm,
             src_v, dst_v, val_v, rows_v, msg_v, pa_v, pb_v, b_v,
             sg0, sg1, sg2, sg3, sg4, ss0, ss1, ss2, ss3, ss4, accum, table_s):
        c = lax.axis_index("c")
        s = lax.axis_index("s")
        w = c * NS + s
        sems_g = (sg0, sg1, sg2, sg3, sg4)
        sems_s = (ss0, ss1, ss2, ss3, ss4)
        tslice = pl.ds(s * ROWS_PER_TILE, ROWS_PER_TILE)

        scope_pro = jax.named_scope("prologue")
        scope_pro.__enter__()
        dz = pltpu.async_copy(zeros_hbm, accum.at[tslice], ss0)
        if fuse_relu:
            da = pltpu.async_copy(table_hbm.at[0, tslice], pa_v, sg0)
            db = pltpu.async_copy(table_hbm.at[1, tslice], pb_v, sg1)
        else:
            dt = pltpu.async_copy(table_hbm.at[tslice], table_s.at[tslice],
                                  sg0)
        d1 = pltpu.async_copy(src_hbm.at[w], src_v, ss1)
        d2 = pltpu.async_copy(dst_hbm.at[w], dst_v, ss2)
        d3 = pltpu.async_copy(val_hbm.at[w], val_v, ss3)
        if fuse_relu:
            pltpu.sync_copy(b_hbm, b_v)
            da.wait()
            db.wait()
            bb = b_v[0, :]

            def relu_row(r4, carry):
                for u in range(4):
                    r = r4 * 4 + u
                    pa_v[r, :] = jnp.maximum(pa_v[r, :] + pb_v[r, :] + bb,
                                             0.0)
                return carry

            lax.fori_loop(0, ROWS_PER_TILE // 4, relu_row, 0)
            pltpu.sync_copy(pa_v, table_s.at[tslice])
        else:
            dt.wait()
        dz.wait()
        d1.wait()
        d2.wait()
        d3.wait()
        plsc.subcore_barrier()
        scope_pro.__exit__(None, None, None)
        scope_main = jax.named_scope("edge_loop")
        gather_src = table_s

        def g_start(k, b):
            for h in range(NI):
                pltpu.async_copy(gather_src.at[dst_v.at[k, h]],
                                 rows_v.at[b, pl.ds(h * CI, CI)], sems_g[b])

        def g_wait(k, b):
            for h in range(NI):
                pltpu.make_async_copy(gather_src.at[dst_v.at[k, h]],
                                      rows_v.at[b, pl.ds(h * CI, CI)],
                                      sems_g[b]).wait()

        def s_start(k, b):
            for h in range(NI):
                pltpu.async_copy(msg_v.at[b, pl.ds(h * CI, CI)],
                                 accum.at[src_v.at[k, h]], sems_s[b],
                                 add=True)

        def s_wait(k, b):
            for h in range(NI):
                pltpu.make_async_copy(msg_v.at[b, pl.ds(h * CI, CI)],
                                      accum.at[src_v.at[k, h]],
                                      sems_s[b]).wait()

        def compute(k, b):
            for g in range(CH // L):
                vv = val_v[k, pl.ds(g * L, L)]
                for j in range(L):
                    i = g * L + j
                    msg_v[b, i, :] = rows_v[b, i, :] * vv[j]

        scope_main.__enter__()
        for b in range(NBUF):
            g_start(b, b)

        def outer(kk, carry):
            for b in range(NBUF):
                k = kk * NBUF + b
                g_wait(k, b)

                @pl.when(k >= NBUF)
                def _():
                    s_wait(k - NBUF, b)

                compute(k, b)
                g_start(k + NBUF, b)
                s_start(k, b)
            return carry

        lax.fori_loop(0, (kch - NBUF) // NBUF, outer, 0)
        for b in range(NBUF):
            k = kch - NBUF + b
            g_wait(k, b)
            s_wait(k - NBUF, b)
            compute(k, b)
            s_start(k, b)
        for b in range(NBUF):
            s_wait(kch - NBUF + b, b)
        scope_main.__exit__(None, None, None)
        with jax.named_scope("writeout"):
            plsc.subcore_barrier()
            pltpu.sync_copy(accum.at[tslice], out_hbm.at[c, tslice])

    prt = ROWS_PER_TILE if fuse_relu else 8
    return pl.kernel(
        body,
        out_type=jax.ShapeDtypeStruct((NC, NP, H), jnp.float32),
        mesh=mesh,
        scratch_types=[
            pltpu.VMEM((kch, NI, CI), jnp.int32),
            pltpu.VMEM((kch, NI, CI), jnp.int32),
            pltpu.VMEM((kch, CH), jnp.float32),
            pltpu.VMEM((NBUF, CH, H), jnp.float32),
            pltpu.VMEM((NBUF, CH, H), jnp.float32),
            pltpu.VMEM((prt, H), jnp.float32),
            pltpu.VMEM((prt, H), jnp.float32),
            pltpu.VMEM((1, H), jnp.float32),
            pltpu.SemaphoreType.DMA,
            pltpu.SemaphoreType.DMA,
            pltpu.SemaphoreType.DMA,
            pltpu.SemaphoreType.DMA,
            pltpu.SemaphoreType.DMA,
            pltpu.SemaphoreType.DMA,
            pltpu.SemaphoreType.DMA,
            pltpu.SemaphoreType.DMA,
            pltpu.SemaphoreType.DMA,
            pltpu.SemaphoreType.DMA,
            pltpu.VMEM_SHARED((NP, H), jnp.float32),
            pltpu.VMEM_SHARED((NP, H), jnp.float32),
        ],
        compiler_params=pltpu.CompilerParams(use_tc_tiling_on_sc=False),
    )


def _mm1_body(x_ref, w_ref, o_ref):
    o_ref[:N, :] = jnp.dot(x_ref[:, :], w_ref[:, :],
                           preferred_element_type=jnp.float32)


def _relu_body(p_ref, b_ref, o_ref):
    o_ref[:, :] = jnp.maximum(p_ref[0] + p_ref[1] + b_ref[:, :], 0.0)


def _out_body(p_ref, w_ref, b_ref, o_ref):
    a = p_ref[0] + p_ref[1]
    t = jnp.dot(a, w_ref[:, :], preferred_element_type=jnp.float32)
    o_ref[:, :] = jax.nn.sigmoid(t + b_ref[:, :])[:N, :]


def kernel(x, lap_indices, lap_values, W1, b1, W2, b2):
    E = lap_values.shape[0]
    src = lap_indices[0].astype(jnp.int32)
    dst = lap_indices[1].astype(jnp.int32)

    kch = -(-E // (NW * CH))
    kch = ((kch + NBUF - 1) // NBUF) * NBUF
    e_pad = NW * kch * CH
    padn = e_pad - E
    if padn:
        pad_idx = (jnp.arange(padn, dtype=jnp.int32) * 61) % N
        src = jnp.concatenate([src, pad_idx])
        dst = jnp.concatenate([dst, pad_idx])
        lap_values = jnp.concatenate(
            [lap_values, jnp.zeros((padn,), jnp.float32)])
    src3 = src.reshape(NW, kch, NI, CI)
    dst3 = dst.reshape(NW, kch, NI, CI)
    val3 = lap_values.reshape(NW, kch, CH)
    zeros = jnp.zeros((ROWS_PER_TILE, H), jnp.float32)

    spmm1 = _make_spmm(kch, False)
    spmm2 = _make_spmm(kch, True)

    t1 = pl.pallas_call(
        _mm1_body,
        out_shape=jax.ShapeDtypeStruct((NP, H), jnp.float32),
    )(x, W1)

    b1r = b1.reshape(1, H)
    p1 = spmm1(t1, b1r, src3, dst3, val3, zeros)
    p2 = spmm2(p1, b1r, src3, dst3, val3, zeros)

    out = pl.pallas_call(
        _out_body,
        out_shape=jax.ShapeDtypeStruct((N, 1), jnp.float32),
    )(p2, W2, b2.reshape(1, 1))
    return out
